# Initial kernel scaffold; baseline (speedup 1.0000x reference)
import jax
import jax.numpy as jnp
from jax.experimental import pallas as pl


def kernel(unique_feature, history_0, history_1, history_2, label_0, label_1, label_2, emb_table, W0, b0, W1, b1, W2, b2):
    raise NotImplementedError("write your pallas kernel here")


# speedup vs baseline: 4.0263x; 4.0263x over previous
import dataclasses
import functools

import jax
import jax.numpy as jnp
from jax import lax
from jax.experimental import pallas as pl
from jax.experimental.pallas import tpu as pltpu
from jax.experimental.pallas import tpu_sc as plsc

_VOCAB = 1000000
_DIM = 64
_U = 100000
_B = 4096
_HIST = 50
_OUT = 6
_EPS = 1e-09

_NC = 2
_NS = 16
_NW = _NC * _NS
_R = 3 * _B
_RPW = _R // _NW
_CH = 8
_IDX_PER = _CH * _HIST
_NCHUNK = _RPW // _CH
_SUBS = ((0, 128), (128, 128), (256, 128), (384, 16))


def _sc_pool(uf, hist_flat, emb):
    mesh = plsc.VectorSubcoreMesh(core_axis_name="c", subcore_axis_name="s")
    cp = pltpu.CompilerParams(use_tc_tiling_on_sc=False)
    if "needs_layout_passes" in pltpu.CompilerParams.__dataclass_fields__:
        cp = dataclasses.replace(cp, needs_layout_passes=False)

    @functools.partial(
        pl.kernel,
        out_type=jax.ShapeDtypeStruct((_R, _DIM), jnp.float32),
        mesh=mesh,
        compiler_params=cp,
        scratch_types=[
            pltpu.VMEM((_U,), jnp.int32),
            pltpu.VMEM((_IDX_PER,), jnp.int32),
            pltpu.VMEM((_IDX_PER,), jnp.int32),
            pltpu.VMEM((_IDX_PER, _DIM), jnp.float32),
            pltpu.VMEM((_CH, _DIM), jnp.float32),
            pltpu.SemaphoreType.DMA,
        ],
    )
    def k(uf_hbm, hist_hbm, emb_hbm, out_hbm, uf_v, hist_v, idx_v, rows_v,
          out_v, gsem):
        wid = lax.axis_index("s") * _NC + lax.axis_index("c")
        pltpu.sync_copy(uf_hbm, uf_v)
        hbase = wid * (_RPW * _HIST)
        obase = wid * _RPW

        @pl.loop(0, _NCHUNK)
        def _(c):
            pltpu.sync_copy(hist_hbm.at[pl.ds(hbase + c * _IDX_PER, _IDX_PER)],
                            hist_v)
            for i in range(_IDX_PER // 16):
                hv = hist_v[pl.ds(i * 16, 16)]
                idx_v[pl.ds(i * 16, 16)] = plsc.load_gather(uf_v, [hv])
            cps = [
                pltpu.make_async_copy(
                    emb_hbm.at[idx_v.at[pl.ds(off, n)]],
                    rows_v.at[pl.ds(off, n)], gsem)
                for off, n in _SUBS
            ]
            for cp in cps:
                cp.start()
            for cp in cps:
                cp.wait()
            for r in range(_CH):
                def body(h, accs, r=r):
                    row = r * _HIST + h
                    return tuple(
                        accs[d] + rows_v[row, pl.ds(d * 16, 16)]
                        for d in range(_DIM // 16))
                accs = lax.fori_loop(
                    0, _HIST, body,
                    tuple(jnp.zeros((16,), jnp.float32)
                          for _ in range(_DIM // 16)))
                for d in range(_DIM // 16):
                    out_v[r, pl.ds(d * 16, 16)] = accs[d]
            pltpu.sync_copy(out_v, out_hbm.at[pl.ds(obase + c * _CH, _CH)])

    return k(uf, hist_flat, emb)


def _tc_head(pooled3, labels_f, Ws, bs):
    def body(p_ref, l_ref, w_ref, b_ref, loss_ref, f1_ref, acc_ref):
        loss_sum = jnp.float32(0.0)
        correct = jnp.float32(0.0)
        ptp = jnp.float32(0.0)
        pfp = jnp.float32(0.0)
        pfn = jnp.float32(0.0)
        ntp = jnp.float32(0.0)
        nfp = jnp.float32(0.0)
        nfn = jnp.float32(0.0)
        for i in range(3):
            x = p_ref[i]
            sq = jnp.sum(x * x, axis=1, keepdims=True)
            normed = x * lax.rsqrt(jnp.maximum(sq, 1e-12))
            logits = jnp.dot(normed, w_ref[i],
                             preferred_element_type=jnp.float32) + b_ref[i]
            pred = jnp.clip(jax.nn.sigmoid(logits), _EPS, 1.0 - _EPS)
            lab = l_ref[i]
            loss = -lab * jnp.log(pred) - (1.0 - lab) * jnp.log(1.0 - pred)
            loss_sum = loss_sum + jnp.sum(loss) * jnp.float32(1.0 / _B)
            pred_label = pred > 0.5
            bool_label = lab == 1.0
            correct = correct + jnp.sum(
                (pred_label == bool_label).astype(jnp.float32))
            ptp = ptp + jnp.sum(
                jnp.logical_and(bool_label, pred_label).astype(jnp.float32))
            pfp = pfp + jnp.sum(jnp.logical_and(
                jnp.logical_not(bool_label), pred_label).astype(jnp.float32))
            pfn = pfn + jnp.sum(jnp.logical_and(
                bool_label, jnp.logical_not(pred_label)).astype(jnp.float32))
            pred_label_n = pred < 0.5
            bool_label_n = lab == 0.0
            ntp = ntp + jnp.sum(jnp.logical_and(
                bool_label_n, pred_label_n).astype(jnp.float32))
            nfp = nfp + jnp.sum(jnp.logical_and(
                jnp.logical_not(bool_label_n), pred_label_n
            ).astype(jnp.float32))
            nfn = nfn + jnp.sum(jnp.logical_and(
                bool_label_n, jnp.logical_not(pred_label_n)
            ).astype(jnp.float32))
        accuracy = correct / jnp.float32(_B * 18)
        pos_recall = ptp / jnp.maximum(_EPS, ptp + pfn)
        pos_precision = ptp / jnp.maximum(_EPS, ptp + pfp)
        pos_f1 = (2 * pos_recall * pos_precision
                  / jnp.maximum(_EPS, pos_recall + pos_precision))
        neg_recall = ntp / jnp.maximum(_EPS, ntp + nfn)
        neg_precision = ntp / jnp.maximum(_EPS, ntp + nfp)
        neg_f1 = (2 * neg_recall * neg_precision
                  / jnp.maximum(_EPS, neg_recall + neg_precision))
        loss_ref[...] = jnp.reshape(loss_sum, (1, 1))
        f1_ref[...] = jnp.reshape((pos_f1 + neg_f1) / 2.0, (1, 1))
        acc_ref[...] = jnp.reshape(accuracy, (1, 1))

    out_shape = [jax.ShapeDtypeStruct((1, 1), jnp.float32)] * 3
    return pl.pallas_call(body, out_shape=out_shape)(
        pooled3, labels_f, Ws, bs)


def kernel(unique_feature, history_0, history_1, history_2, label_0, label_1,
           label_2, emb_table, W0, b0, W1, b1, W2, b2):
    uf = unique_feature.astype(jnp.int32)
    hist_flat = jnp.concatenate(
        [history_0, history_1, history_2], axis=0).reshape(-1)
    pooled = _sc_pool(uf, hist_flat, emb_table)
    pooled3 = pooled.reshape(3, _B, _DIM)
    labels_f = jnp.stack([label_0, label_1, label_2]).astype(jnp.float32)
    Ws = jnp.stack([W0, W1, W2])
    bs = jnp.stack([b0, b1, b2]).reshape(3, 1, _OUT)
    loss, f1, acc = _tc_head(pooled3, labels_f, Ws, bs)
    return loss[0, 0], f1[0, 0], acc[0, 0]


# speedup vs baseline: 6.4869x; 1.6111x over previous
import dataclasses
import functools

import jax
import jax.numpy as jnp
from jax import lax
from jax.experimental import pallas as pl
from jax.experimental.pallas import tpu as pltpu
from jax.experimental.pallas import tpu_sc as plsc

_VOCAB = 1000000
_DIM = 64
_U = 100000
_B = 4096
_HIST = 50
_OUT = 6
_EPS = 1e-09

_NC = 2
_NS = 16
_NW = _NC * _NS
_R = 3 * _B
_RPW = _R // _NW
_CH = 8
_IDX_PER = _CH * _HIST
_NCHUNK = _RPW // _CH
_SUBS = ((0, 128), (128, 128), (256, 128), (384, 16))

_PB = 2048
_PG = -(-(_VOCAB // 2) // _PB) + 0
_PG = (_VOCAB + 2 * _PB - 1) // (2 * _PB)
_PROWS = _PG * _PB
_NIN = (_VOCAB + _PB - 1) // _PB


def _tc_pack(embT):
    def body(x1_ref, x2_ref, o_ref):
        o_ref[...] = jnp.concatenate(
            [x1_ref[...], x2_ref[...]], axis=0).T

    return pl.pallas_call(
        body,
        grid=(_PG,),
        in_specs=[
            pl.BlockSpec((_DIM, _PB),
                         lambda b: (0, jnp.minimum(2 * b, _NIN - 1))),
            pl.BlockSpec((_DIM, _PB),
                         lambda b: (0, jnp.minimum(2 * b + 1, _NIN - 1))),
        ],
        out_specs=pl.BlockSpec((_PB, 128), lambda b: (b, 0)),
        out_shape=jax.ShapeDtypeStruct((_PROWS, 128), jnp.float32),
    )(embT, embT)


def _sc_pool(uf, hist_flat, emb):
    mesh = plsc.VectorSubcoreMesh(core_axis_name="c", subcore_axis_name="s")
    cp = pltpu.CompilerParams(use_tc_tiling_on_sc=False)
    if "needs_layout_passes" in pltpu.CompilerParams.__dataclass_fields__:
        cp = dataclasses.replace(cp, needs_layout_passes=False)

    @functools.partial(
        pl.kernel,
        out_type=jax.ShapeDtypeStruct((_R, _DIM), jnp.float32),
        mesh=mesh,
        compiler_params=cp,
        scratch_types=[
            pltpu.VMEM((_U,), jnp.int32),
            pltpu.VMEM((_IDX_PER,), jnp.int32),
            pltpu.VMEM((_IDX_PER,), jnp.int32),
            pltpu.VMEM((_IDX_PER, _DIM), jnp.float32),
            pltpu.VMEM((_CH, _DIM), jnp.float32),
            pltpu.SemaphoreType.DMA,
        ],
    )
    def k(uf_hbm, hist_hbm, emb_hbm, out_hbm, uf_v, hist_v, idx_v,
          rows_v, out_v, gsem):
        wid = lax.axis_index("s") * _NC + lax.axis_index("c")
        pltpu.sync_copy(uf_hbm, uf_v)
        hbase = wid * (_RPW * _HIST)
        obase = wid * _RPW

        @pl.loop(0, _NCHUNK)
        def _(c):
            pltpu.sync_copy(hist_hbm.at[pl.ds(hbase + c * _IDX_PER, _IDX_PER)],
                            hist_v)
            for i in range(_IDX_PER // 16):
                hv = hist_v[pl.ds(i * 16, 16)]
                fv = plsc.load_gather(uf_v, [hv])
                idx_v[pl.ds(i * 16, 16)] = lax.bitwise_or(
                    lax.bitwise_or(
                        lax.shift_left(lax.shift_right_logical(fv, 12), 12),
                        lax.shift_left(
                            lax.bitwise_and(
                                fv, jnp.full((16,), _PB - 1, jnp.int32)), 1)),
                    lax.bitwise_and(lax.shift_right_logical(fv, 11),
                                    jnp.full((16,), 1, jnp.int32)))
            cps = [
                pltpu.make_async_copy(
                    emb_hbm.at[idx_v.at[pl.ds(off, n)]],
                    rows_v.at[pl.ds(off, n)], gsem)
                for off, n in _SUBS
            ]
            for cp in cps:
                cp.start()
            for cp in cps:
                cp.wait()
            for r in range(_CH):
                def body(h, accs, r=r):
                    row = r * _HIST + h
                    return tuple(
                        accs[d] + rows_v[row, pl.ds(d * 16, 16)]
                        for d in range(_DIM // 16))
                accs = lax.fori_loop(
                    0, _HIST, body,
                    tuple(jnp.zeros((16,), jnp.float32)
                          for _ in range(_DIM // 16)))
                for d in range(_DIM // 16):
                    out_v[r, pl.ds(d * 16, 16)] = accs[d]
            pltpu.sync_copy(out_v, out_hbm.at[pl.ds(obase + c * _CH, _CH)])

    return k(uf, hist_flat, emb)


def _tc_head(pooled3, labels_f, Ws, bs):
    def body(p_ref, l_ref, w_ref, b_ref, loss_ref, f1_ref, acc_ref):
        loss_sum = jnp.float32(0.0)
        correct = jnp.float32(0.0)
        ptp = jnp.float32(0.0)
        pfp = jnp.float32(0.0)
        pfn = jnp.float32(0.0)
        ntp = jnp.float32(0.0)
        nfp = jnp.float32(0.0)
        nfn = jnp.float32(0.0)
        for i in range(3):
            x = p_ref[i]
            sq = jnp.sum(x * x, axis=1, keepdims=True)
            normed = x * lax.rsqrt(jnp.maximum(sq, 1e-12))
            logits = jnp.dot(normed, w_ref[i],
                             preferred_element_type=jnp.float32) + b_ref[i]
            pred = jnp.clip(jax.nn.sigmoid(logits), _EPS, 1.0 - _EPS)
            lab = l_ref[i]
            loss = -lab * jnp.log(pred) - (1.0 - lab) * jnp.log(1.0 - pred)
            loss_sum = loss_sum + jnp.sum(loss) * jnp.float32(1.0 / _B)
            pred_label = pred > 0.5
            bool_label = lab == 1.0
            correct = correct + jnp.sum(
                (pred_label == bool_label).astype(jnp.float32))
            ptp = ptp + jnp.sum(
                jnp.logical_and(bool_label, pred_label).astype(jnp.float32))
            pfp = pfp + jnp.sum(jnp.logical_and(
                jnp.logical_not(bool_label), pred_label).astype(jnp.float32))
            pfn = pfn + jnp.sum(jnp.logical_and(
                bool_label, jnp.logical_not(pred_label)).astype(jnp.float32))
            pred_label_n = pred < 0.5
            bool_label_n = lab == 0.0
            ntp = ntp + jnp.sum(jnp.logical_and(
                bool_label_n, pred_label_n).astype(jnp.float32))
            nfp = nfp + jnp.sum(jnp.logical_and(
                jnp.logical_not(bool_label_n), pred_label_n
            ).astype(jnp.float32))
            nfn = nfn + jnp.sum(jnp.logical_and(
                bool_label_n, jnp.logical_not(pred_label_n)
            ).astype(jnp.float32))
        accuracy = correct / jnp.float32(_B * 18)
        pos_recall = ptp / jnp.maximum(_EPS, ptp + pfn)
        pos_precision = ptp / jnp.maximum(_EPS, ptp + pfp)
        pos_f1 = (2 * pos_recall * pos_precision
                  / jnp.maximum(_EPS, pos_recall + pos_precision))
        neg_recall = ntp / jnp.maximum(_EPS, ntp + nfn)
        neg_precision = ntp / jnp.maximum(_EPS, ntp + nfp)
        neg_f1 = (2 * neg_recall * neg_precision
                  / jnp.maximum(_EPS, neg_recall + neg_precision))
        loss_ref[...] = jnp.reshape(loss_sum, (1, 1))
        f1_ref[...] = jnp.reshape((pos_f1 + neg_f1) / 2.0, (1, 1))
        acc_ref[...] = jnp.reshape(accuracy, (1, 1))

    out_shape = [jax.ShapeDtypeStruct((1, 1), jnp.float32)] * 3
    return pl.pallas_call(body, out_shape=out_shape)(
        pooled3, labels_f, Ws, bs)


def kernel(unique_feature, history_0, history_1, history_2, label_0, label_1,
           label_2, emb_table, W0, b0, W1, b1, W2, b2):
    uf = unique_feature.astype(jnp.int32)
    hist_flat = jnp.concatenate(
        [history_0, history_1, history_2], axis=0).reshape(-1)
    packed = _tc_pack(emb_table.T).reshape(2 * _PROWS, _DIM)
    pooled = _sc_pool(uf, hist_flat, packed)
    pooled3 = pooled.reshape(3, _B, _DIM)
    labels_f = jnp.stack([label_0, label_1, label_2]).astype(jnp.float32)
    Ws = jnp.stack([W0, W1, W2])
    bs = jnp.stack([b0, b1, b2]).reshape(3, 1, _OUT)
    loss, f1, acc = _tc_head(pooled3, labels_f, Ws, bs)
    return loss[0, 0], f1[0, 0], acc[0, 0]


# speedup vs baseline: 11.0611x; 1.7052x over previous
import dataclasses
import functools

import jax
import jax.numpy as jnp
from jax import lax
from jax.experimental import pallas as pl
from jax.experimental.pallas import tpu as pltpu
from jax.experimental.pallas import tpu_sc as plsc

_VOCAB = 1000000
_DIM = 64
_U = 100000
_B = 4096
_HIST = 50
_OUT = 6
_EPS = 1e-09

_NC = 2
_NS = 16
_NW = _NC * _NS
_R = 3 * _B
_RPW = _R // _NW
_IPW = _RPW * _HIST
_CH = 16
_IDX_PER = _CH * _HIST
_NCHUNK = _RPW // _CH
_SUBS = tuple((o, min(128, _IDX_PER - o)) for o in range(0, _IDX_PER, 128))

_PB = 8192
_SH = 14
_PG = (_VOCAB + 2 * _PB - 1) // (2 * _PB)
_PROWS = _PG * _PB
_NIN = (_VOCAB + _PB - 1) // _PB


def _tc_pack(embT):
    def body(x1_ref, x2_ref, o_ref):
        o_ref[...] = jnp.concatenate(
            [x1_ref[...], x2_ref[...]], axis=0).T

    return pl.pallas_call(
        body,
        grid=(_PG,),
        in_specs=[
            pl.BlockSpec((_DIM, _PB),
                         lambda b: (0, jnp.minimum(2 * b, _NIN - 1))),
            pl.BlockSpec((_DIM, _PB),
                         lambda b: (0, jnp.minimum(2 * b + 1, _NIN - 1))),
        ],
        out_specs=pl.BlockSpec((_PB, 128), lambda b: (b, 0)),
        out_shape=jax.ShapeDtypeStruct((_PROWS, 128), jnp.float32),
    )(embT, embT)


def _lin_ids(fv):
    return lax.bitwise_or(
        lax.bitwise_or(
            lax.shift_left(lax.shift_right_logical(fv, _SH), _SH),
            lax.shift_left(
                lax.bitwise_and(fv, jnp.full((16,), _PB - 1, jnp.int32)), 1)),
        lax.bitwise_and(lax.shift_right_logical(fv, _SH - 1),
                        jnp.full((16,), 1, jnp.int32)))


def _sc_compiler_params():
    cp = pltpu.CompilerParams(use_tc_tiling_on_sc=False)
    if "needs_layout_passes" in pltpu.CompilerParams.__dataclass_fields__:
        cp = dataclasses.replace(cp, needs_layout_passes=False)
    return cp


def _sc_idx(uf, hist_flat):
    mesh = plsc.VectorSubcoreMesh(core_axis_name="c", subcore_axis_name="s")
    chunk = 2400
    nch = _IPW // chunk

    @functools.partial(
        pl.kernel,
        out_type=jax.ShapeDtypeStruct((_R * _HIST,), jnp.int32),
        mesh=mesh,
        compiler_params=_sc_compiler_params(),
        scratch_types=[
            pltpu.VMEM((_U,), jnp.int32),
            pltpu.VMEM((chunk,), jnp.int32),
        ],
    )
    def k(uf_hbm, hist_hbm, out_hbm, uf_v, hist_v):
        wid = lax.axis_index("s") * _NC + lax.axis_index("c")
        base = wid * _IPW
        pltpu.sync_copy(uf_hbm, uf_v)

        @pl.loop(0, nch)
        def _(c):
            pltpu.sync_copy(hist_hbm.at[pl.ds(base + c * chunk, chunk)],
                            hist_v)
            for i in range(chunk // 16):
                hv = hist_v[pl.ds(i * 16, 16)]
                hist_v[pl.ds(i * 16, 16)] = _lin_ids(
                    plsc.load_gather(uf_v, [hv]))
            pltpu.sync_copy(hist_v,
                            out_hbm.at[pl.ds(base + c * chunk, chunk)])

    return k(uf, hist_flat)


def _sc_pool(lin_idx, emb):
    mesh = plsc.VectorSubcoreMesh(core_axis_name="c", subcore_axis_name="s")

    @functools.partial(
        pl.kernel,
        out_type=jax.ShapeDtypeStruct((_R, _DIM), jnp.float32),
        mesh=mesh,
        compiler_params=_sc_compiler_params(),
        scratch_types=[
            pltpu.VMEM((_IPW,), jnp.int32),
            pltpu.VMEM((_IDX_PER, _DIM), jnp.float32),
            pltpu.VMEM((_IDX_PER, _DIM), jnp.float32),
            pltpu.VMEM((2 * _CH, _DIM), jnp.float32),
            pltpu.SemaphoreType.DMA,
            pltpu.SemaphoreType.DMA,
        ],
    )
    def k(lin_hbm, emb_hbm, out_hbm, lin_v, rows0, rows1, out_v, sem0, sem1):
        wid = lax.axis_index("s") * _NC + lax.axis_index("c")
        base = wid * _IPW
        obase = wid * _RPW
        pltpu.sync_copy(lin_hbm.at[pl.ds(base, _IPW)], lin_v)

        def fire(c, rows, sem):
            for off, n in _SUBS:
                pltpu.make_async_copy(
                    emb_hbm.at[lin_v.at[pl.ds(c * _IDX_PER + off, n)]],
                    rows.at[pl.ds(off, n)], sem).start()

        def drain(c, rows, sem):
            for off, n in _SUBS:
                pltpu.make_async_copy(
                    emb_hbm.at[lin_v.at[pl.ds(c * _IDX_PER + off, n)]],
                    rows.at[pl.ds(off, n)], sem).wait()

        def accumulate(rows, out_off):
            for r in range(_CH):
                def body(h, accs, r=r):
                    row = r * _HIST + 2 * h
                    return tuple(
                        accs[d]
                        + rows[row, pl.ds(d * 16, 16)]
                        + rows[row + 1, pl.ds(d * 16, 16)]
                        for d in range(_DIM // 16))
                accs = lax.fori_loop(
                    0, _HIST // 2, body,
                    tuple(jnp.zeros((16,), jnp.float32)
                          for _ in range(_DIM // 16)))
                for d in range(_DIM // 16):
                    out_v[out_off + r, pl.ds(d * 16, 16)] = accs[d]

        fire(0, rows0, sem0)

        @pl.loop(0, _NCHUNK // 2)
        def _(cc):
            c0 = 2 * cc
            fire(c0 + 1, rows1, sem1)
            drain(c0, rows0, sem0)
            accumulate(rows0, 0)

            @pl.when(cc < _NCHUNK // 2 - 1)
            def _():
                fire(c0 + 2, rows0, sem0)

            drain(c0 + 1, rows1, sem1)
            accumulate(rows1, _CH)
            pltpu.sync_copy(out_v,
                            out_hbm.at[pl.ds(obase + c0 * _CH, 2 * _CH)])

    return k(lin_idx, emb)


def _tc_head(pooled3, labels_f, Ws, bs):
    def body(p_ref, l_ref, w_ref, b_ref, loss_ref, f1_ref, acc_ref):
        loss_sum = jnp.float32(0.0)
        correct = jnp.float32(0.0)
        ptp = jnp.float32(0.0)
        pfp = jnp.float32(0.0)
        pfn = jnp.float32(0.0)
        ntp = jnp.float32(0.0)
        nfp = jnp.float32(0.0)
        nfn = jnp.float32(0.0)
        for i in range(3):
            x = p_ref[i]
            sq = jnp.sum(x * x, axis=1, keepdims=True)
            normed = x * lax.rsqrt(jnp.maximum(sq, 1e-12))
            logits = jnp.dot(normed, w_ref[i],
                             preferred_element_type=jnp.float32) + b_ref[i]
            pred = jnp.clip(jax.nn.sigmoid(logits), _EPS, 1.0 - _EPS)
            lab = l_ref[i]
            loss = -lab * jnp.log(pred) - (1.0 - lab) * jnp.log(1.0 - pred)
            loss_sum = loss_sum + jnp.sum(loss) * jnp.float32(1.0 / _B)
            pred_label = pred > 0.5
            bool_label = lab == 1.0
            correct = correct + jnp.sum(
                (pred_label == bool_label).astype(jnp.float32))
            ptp = ptp + jnp.sum(
                jnp.logical_and(bool_label, pred_label).astype(jnp.float32))
            pfp = pfp + jnp.sum(jnp.logical_and(
                jnp.logical_not(bool_label), pred_label).astype(jnp.float32))
            pfn = pfn + jnp.sum(jnp.logical_and(
                bool_label, jnp.logical_not(pred_label)).astype(jnp.float32))
            pred_label_n = pred < 0.5
            bool_label_n = lab == 0.0
            ntp = ntp + jnp.sum(jnp.logical_and(
                bool_label_n, pred_label_n).astype(jnp.float32))
            nfp = nfp + jnp.sum(jnp.logical_and(
                jnp.logical_not(bool_label_n), pred_label_n
            ).astype(jnp.float32))
            nfn = nfn + jnp.sum(jnp.logical_and(
                bool_label_n, jnp.logical_not(pred_label_n)
            ).astype(jnp.float32))
        accuracy = correct / jnp.float32(_B * 18)
        pos_recall = ptp / jnp.maximum(_EPS, ptp + pfn)
        pos_precision = ptp / jnp.maximum(_EPS, ptp + pfp)
        pos_f1 = (2 * pos_recall * pos_precision
                  / jnp.maximum(_EPS, pos_recall + pos_precision))
        neg_recall = ntp / jnp.maximum(_EPS, ntp + nfn)
        neg_precision = ntp / jnp.maximum(_EPS, ntp + nfp)
        neg_f1 = (2 * neg_recall * neg_precision
                  / jnp.maximum(_EPS, neg_recall + neg_precision))
        loss_ref[...] = jnp.reshape(loss_sum, (1, 1))
        f1_ref[...] = jnp.reshape((pos_f1 + neg_f1) / 2.0, (1, 1))
        acc_ref[...] = jnp.reshape(accuracy, (1, 1))

    out_shape = [jax.ShapeDtypeStruct((1, 1), jnp.float32)] * 3
    return pl.pallas_call(body, out_shape=out_shape)(
        pooled3, labels_f, Ws, bs)


def kernel(unique_feature, history_0, history_1, history_2, label_0, label_1,
           label_2, emb_table, W0, b0, W1, b1, W2, b2):
    uf = unique_feature.astype(jnp.int32)
    hist_flat = jnp.concatenate(
        [history_0, history_1, history_2], axis=0).reshape(-1)
    packed = _tc_pack(emb_table.T).reshape(2 * _PROWS, _DIM)
    lin = _sc_idx(uf, hist_flat)
    pooled = _sc_pool(lin, packed)
    pooled3 = pooled.reshape(3, _B, _DIM)
    labels_f = jnp.stack([label_0, label_1, label_2]).astype(jnp.float32)
    Ws = jnp.stack([W0, W1, W2])
    bs = jnp.stack([b0, b1, b2]).reshape(3, 1, _OUT)
    loss, f1, acc = _tc_head(pooled3, labels_f, Ws, bs)
    return loss[0, 0], f1[0, 0], acc[0, 0]
